# baseline (speedup 1.0000x reference)
import functools

import jax
import jax.numpy as jnp
from jax import lax
from jax.experimental import pallas as pl
from jax.experimental.pallas import tpu as pltpu
from jax.experimental.pallas import tpu_sc as plsc

NC, NS, LANES = 2, 16, 16
NW = NC * NS
HID = 64
CH = 16384
BLK = 256
SEC0, SEC1, SEC2 = 4 * CH, 49 * CH, 49 * CH
OFF0, OFF1, OFF2 = 0, SEC0, SEC0 + SEC1
RPAD = SEC0 + SEC1 + SEC2
NCHUNK = RPAD // CH
BIG = 2 ** 30
HP = 50176


def _mesh():
    return plsc.VectorSubcoreMesh(core_axis_name="c", subcore_axis_name="s",
                                  num_cores=NC, num_subcores=NS)


def _bget(bvm, idx):
    grp = idx >> 4
    off = idx & 15
    v = bvm[pl.ds(grp * 16, 16)]
    lane = lax.iota(jnp.int32, 16)
    return jnp.max(jnp.where(lane == off, v, jnp.int32(-2147483647)))


def _make_segsum(nchunk, use_b):
    out_rows = nchunk * CH

    def body(E, dstS, srcAS, srcBS, bounds, A,
             bvm, dbuf, abuf, bbuf, sbuf, bufA, bufB, prod, acc,
             sem_i, sem_g, sem_s):
        q = lax.axis_index("c")
        t = lax.axis_index("s")
        pltpu.sync_copy(bounds, bvm)

        def chunk_body(i, carry):
            c = i * NC + q
            base = c * CH
            for r in range(4):
                off = t * (CH // NS) + r * BLK
                pltpu.sync_copy(E.at[pl.ds(base + off, BLK)], prod)
                pltpu.sync_copy(prod, acc.at[pl.ds(off, BLK)])
            plsc.subcore_barrier()
            b0 = _bget(bvm, c)
            b1 = _bget(bvm, c + 1)
            blo = b0 >> 8
            bhi = (b1 + (BLK - 1)) >> 8
            nb = bhi - blo
            lo = blo + ((t * nb) >> 4)
            hi = blo + (((t + 1) * nb) >> 4)

            def blk_body(bi, carry2):
                r2 = bi * 2
                d1 = pltpu.async_copy(dstS.at[pl.ds(r2, 2)], dbuf, sem_i)
                d2 = pltpu.async_copy(srcAS.at[pl.ds(r2, 2)], abuf, sem_i)
                if use_b:
                    d3 = pltpu.async_copy(srcBS.at[pl.ds(r2, 2)], bbuf, sem_i)
                d1.wait()
                d2.wait()
                if use_b:
                    d3.wait()
                gs = []
                for j in range(2):
                    gs.append(pltpu.async_copy(
                        E.at[abuf.at[j]], bufA.at[pl.ds(j * 128, 128)], sem_g))
                    if use_b:
                        gs.append(pltpu.async_copy(
                            E.at[bbuf.at[j]], bufB.at[pl.ds(j * 128, 128)], sem_g))
                for j in range(2):
                    for g in range(8):
                        cs = pl.ds(g * 16, 16)
                        dloc = dbuf[j, cs] - base
                        msk = (dloc >= 0) & (dloc < CH)
                        sbuf[j, cs] = jnp.where(msk, dloc, jnp.int32(CH))
                for gd in gs:
                    gd.wait()
                if use_b:
                    def prow(ii, cc2):
                        for rr in range(4):
                            row = ii * 4 + rr
                            for k in range(4):
                                cs = pl.ds(k * 16, 16)
                                prod[row, cs] = bufA[row, cs] * bufB[row, cs]
                        return cc2
                    lax.fori_loop(0, BLK // 4, prow, 0)
                    src = prod
                else:
                    src = bufA
                s1 = pltpu.async_copy(src.at[pl.ds(0, 128)],
                                      acc.at[sbuf.at[0]], sem_s, add=True)
                s2 = pltpu.async_copy(src.at[pl.ds(128, 128)],
                                      acc.at[sbuf.at[1]], sem_s, add=True)
                s1.wait()
                s2.wait()
                return carry2

            lax.fori_loop(lo, hi, blk_body, 0)
            plsc.subcore_barrier()
            for r in range(4):
                off = t * (CH // NS) + r * BLK
                pltpu.sync_copy(acc.at[pl.ds(off, BLK)], prod)
                pltpu.sync_copy(prod, A.at[pl.ds(base + off, BLK)])
            return carry

        lax.fori_loop(0, nchunk // NC, chunk_body, 0)

    return pl.kernel(
        body,
        out_type=jax.ShapeDtypeStruct((out_rows, HID), jnp.float32),
        mesh=_mesh(),
        compiler_params=pltpu.CompilerParams(use_tc_tiling_on_sc=False, needs_layout_passes=False),
        scratch_types=[
            pltpu.VMEM((112,), jnp.int32),
            pltpu.VMEM((2, 128), jnp.int32),
            pltpu.VMEM((2, 128), jnp.int32),
            pltpu.VMEM((2, 128), jnp.int32),
            pltpu.VMEM((2, 128), jnp.int32),
            pltpu.VMEM((BLK, HID), jnp.float32),
            pltpu.VMEM((BLK, HID), jnp.float32),
            pltpu.VMEM((BLK, HID), jnp.float32),
            pltpu.VMEM_SHARED((CH + 8, HID), jnp.float32),
            pltpu.SemaphoreType.DMA,
            pltpu.SemaphoreType.DMA,
            pltpu.SemaphoreType.DMA,
        ],
    )


def _make_build():
    NB0 = HP // BLK
    NB12 = 800000 // BLK

    def body(h, g1, g2, emb, E, embv, ibuf, buf, sem_g):
        q = lax.axis_index("c")
        t = lax.axis_index("s")
        w = q * NS + t
        pltpu.sync_copy(emb, embv)

        def c0(i, cc):
            r0 = (w + i * NW) * BLK
            pltpu.sync_copy(h.at[pl.ds(r0, BLK)], buf)
            pltpu.sync_copy(buf, E.at[pl.ds(OFF0 + r0, BLK)])
            return cc
        lax.fori_loop(0, (NB0 - w + NW - 1) >> 5, c0, 0)

        def sec(gref, erow, off):
            def cs_body(i, cc):
                bi = w + i * NW
                pltpu.sync_copy(gref.at[pl.ds(bi * 2, 2)], ibuf)
                ga = pltpu.async_copy(h.at[ibuf.at[0]],
                                      buf.at[pl.ds(0, 128)], sem_g)
                gb = pltpu.async_copy(h.at[ibuf.at[1]],
                                      buf.at[pl.ds(128, 128)], sem_g)
                ga.wait()
                gb.wait()

                def addrow(ii, cc2):
                    for rr in range(4):
                        row = ii * 4 + rr
                        for k in range(4):
                            cs = pl.ds(k * 16, 16)
                            buf[row, cs] = buf[row, cs] + embv[erow, cs]
                    return cc2
                lax.fori_loop(0, BLK // 4, addrow, 0)
                pltpu.sync_copy(buf, E.at[pl.ds(off + bi * BLK, BLK)])
                return cc
            lax.fori_loop(0, (NB12 - w + NW - 1) >> 5, cs_body, 0)

        sec(g1, 0, OFF1)
        sec(g2, 1, OFF2)

    return pl.kernel(
        body,
        out_type=jax.ShapeDtypeStruct((RPAD, HID), jnp.float32),
        mesh=_mesh(),
        compiler_params=pltpu.CompilerParams(use_tc_tiling_on_sc=False, needs_layout_passes=False),
        scratch_types=[
            pltpu.VMEM((2, HID), jnp.float32),
            pltpu.VMEM((2, 128), jnp.int32),
            pltpu.VMEM((BLK, HID), jnp.float32),
            pltpu.SemaphoreType.DMA,
        ],
    )


def _h_proj(x, W_proj, b_proj):
    RB = 3584
    xp = jnp.pad(x, ((0, HP - x.shape[0]), (0, 0)))

    def hk(x_ref, w_ref, b_ref, o_ref):
        o_ref[...] = x_ref[...] * w_ref[...] + b_ref[...]

    return pl.pallas_call(
        hk,
        grid=(HP // RB,),
        in_specs=[
            pl.BlockSpec((RB, 1), lambda i: (i, 0)),
            pl.BlockSpec((1, HID), lambda i: (0, 0)),
            pl.BlockSpec((1, HID), lambda i: (0, 0)),
        ],
        out_specs=pl.BlockSpec((RB, HID), lambda i: (i, 0)),
        out_shape=jax.ShapeDtypeStruct((HP, HID), jnp.float32),
    )(xp, W_proj, b_proj.reshape(1, HID))


def _elu(z):
    return jnp.where(z > 0, z, jnp.exp(jnp.minimum(z, 0.0)) - 1.0)


def _mlp3(A, W1, b1, W2, b2):
    RB = 2048
    C0, C1 = SEC0 // RB, (SEC0 + SEC1) // RB

    def smap(i):
        s = (i >= C0).astype(jnp.int32) + (i >= C1).astype(jnp.int32)
        return (s, 0, 0)

    def body(a_ref, w1, b1r, w2, b2r, o_ref):
        z = jnp.dot(a_ref[...], w1[0], preferred_element_type=jnp.float32) + b1r[0]
        z = _elu(z)
        o_ref[...] = jnp.dot(z, w2[0], preferred_element_type=jnp.float32) + b2r[0]

    return pl.pallas_call(
        body,
        grid=(RPAD // RB,),
        in_specs=[
            pl.BlockSpec((RB, HID), lambda i: (i, 0)),
            pl.BlockSpec((1, HID, HID), smap),
            pl.BlockSpec((1, 1, HID), smap),
            pl.BlockSpec((1, HID, HID), smap),
            pl.BlockSpec((1, 1, HID), smap),
        ],
        out_specs=pl.BlockSpec((RB, HID), lambda i: (i, 0)),
        out_shape=jax.ShapeDtypeStruct((RPAD, HID), jnp.float32),
    )(A, W1.reshape(3, HID, HID), b1.reshape(3, 1, HID),
      W2.reshape(3, HID, HID), b2.reshape(3, 1, HID))


def _pool_mlp(P, Wp1, bp1, Wp2, bp2):
    RB = 4096

    def body(p_ref, w1, b1r, w2, b2r, o_ref):
        z = jnp.dot(p_ref[...], w1[...], preferred_element_type=jnp.float32) + b1r[...]
        z = _elu(z)
        o_ref[...] = jnp.dot(z, w2[...], preferred_element_type=jnp.float32) + b2r[...]

    return pl.pallas_call(
        body,
        grid=(SEC0 // RB,),
        in_specs=[
            pl.BlockSpec((RB, HID), lambda i: (i, 0)),
            pl.BlockSpec((HID, HID // 2), lambda i: (0, 0)),
            pl.BlockSpec((1, HID // 2), lambda i: (0, 0)),
            pl.BlockSpec((HID // 2, 1), lambda i: (0, 0)),
            pl.BlockSpec((1, 1), lambda i: (0, 0)),
        ],
        out_specs=pl.BlockSpec((RB, 1), lambda i: (i, 0)),
        out_shape=jax.ShapeDtypeStruct((SEC0, 1), jnp.float32),
    )(P, Wp1, bp1.reshape(1, -1), Wp2, bp2.reshape(1, 1))


def _sorted_entries(dst, srcA, srcB):
    k = dst.shape[0]
    kp = ((k + BLK - 1) // BLK) * BLK
    pad = kp - k
    dst_s, a_s, b_s = lax.sort((dst, srcA, srcB), dimension=0, num_keys=1)
    if pad:
        dst_s = jnp.concatenate([dst_s, jnp.full((pad,), BIG, jnp.int32)])
        a_s = jnp.concatenate([a_s, jnp.zeros((pad,), jnp.int32)])
        b_s = jnp.concatenate([b_s, jnp.zeros((pad,), jnp.int32)])
    else:
        dst_s = jnp.concatenate([dst_s, jnp.full((BLK,), BIG, jnp.int32)])
        a_s = jnp.concatenate([a_s, jnp.zeros((BLK,), jnp.int32)])
        b_s = jnp.concatenate([b_s, jnp.zeros((BLK,), jnp.int32)])
    return dst_s, a_s, b_s


def _bounds(dst_sorted_padded, nchunk):
    edges = jnp.arange(nchunk + 1, dtype=jnp.int32) * CH
    b = jnp.searchsorted(dst_sorted_padded, edges, side="left").astype(jnp.int32)
    return jnp.pad(b, (0, 112 - (nchunk + 1)), constant_values=b[nchunk])


def _r128(a):
    return a.reshape(a.shape[0] // 128, 128)


def kernel(x, triangle_0_1_1, triangle_1_1_1, triangle_1_1_2, triangle_1_2_2,
           triangle_2_2_2, inverse_edge_1, inverse_edge_2, edge_index0,
           edge_index, edge_index2, num_nodes, W_proj, b_proj, emb,
           ker_W1, ker_b1, ker_W2, ker_b2, Wp1, bp1, Wp2, bp2):
    N = x.shape[0]
    E1 = edge_index.shape[1]
    E2 = edge_index2.shape[1]
    t011, t111 = triangle_0_1_1, triangle_1_1_1
    t112, t122, t222 = triangle_1_1_2, triangle_1_2_2, triangle_2_2_2
    i1, i2 = inverse_edge_1, inverse_edge_2
    g1, g2 = edge_index[1], edge_index2[1]
    o1, o2 = jnp.int32(OFF1), jnp.int32(OFF2)

    cc = jnp.concatenate
    dst = cc([t011[0], o1 + t011[1], o1 + t011[2],
              o1 + t111[0], o1 + t111[1], o1 + t111[2],
              o2 + t112[2], o1 + t112[0], o1 + t112[1],
              o1 + t122[0], o2 + t122[1], o2 + t122[2],
              o2 + t222[0], o2 + t222[1], o2 + t222[2]])
    sa = cc([o1 + t011[1], t011[0], t011[0],
             o1 + t111[1], o1 + t111[2], o1 + t111[0],
             o1 + t112[0], o1 + t112[1], o1 + i1[t112[0]],
             o2 + t122[1], o1 + t122[0], o2 + t122[1],
             o2 + t222[1], o2 + t222[2], o2 + t222[0]])
    sb = cc([o1 + t011[2], o1 + i1[t011[2]], o1 + i1[t011[1]],
             o1 + i1[t111[2]], o1 + i1[t111[0]], o1 + i1[t111[1]],
             o1 + t112[1], o2 + i2[t112[2]], o2 + t112[2],
             o2 + i2[t122[2]], o2 + t122[2], o1 + i1[t122[0]],
             o2 + i2[t222[2]], o2 + i2[t222[0]], o2 + i2[t222[1]]])
    dst_s, sa_s, sb_s = _sorted_entries(dst.astype(jnp.int32),
                                        sa.astype(jnp.int32),
                                        sb.astype(jnp.int32))
    bnd = _bounds(dst_s, NCHUNK)
    dst_s, sa_s, sb_s = _r128(dst_s), _r128(sa_s), _r128(sb_s)

    dstp = cc([g1, g2]).astype(jnp.int32)
    srcp = cc([o1 + jnp.arange(E1, dtype=jnp.int32),
               o2 + jnp.arange(E2, dtype=jnp.int32)])
    dstp_s, srcp_s, _unused = _sorted_entries(dstp, srcp, srcp)
    bndp = _bounds(dstp_s, SEC0 // CH)
    dstp_s, srcp_s = _r128(dstp_s), _r128(srcp_s)

    h = _h_proj(x, W_proj, b_proj)
    build = _make_build()
    E = build(h, _r128(g1.astype(jnp.int32)), _r128(g2.astype(jnp.int32)), emb)

    dsts = dst.astype(jnp.int32)
    sas = sa.astype(jnp.int32)
    sbs = sb.astype(jnp.int32)
    for l in range(ker_W1.shape[0]):
        A = E.at[dsts].add(E[sas] * E[sbs], mode="drop",
                           indices_are_sorted=False, unique_indices=False)
        E = _mlp3(A, ker_W1[l], ker_b1[l], ker_W2[l], ker_b2[l])

    P = E[:SEC0].at[dstp].add(E[srcp], mode="drop")
    out = _pool_mlp(P, Wp1, bp1, Wp2, bp2)
    return out[:N, 0]
